# Initial kernel scaffold; baseline (speedup 1.0000x reference)
import jax
import jax.numpy as jnp
from jax.experimental import pallas as pl


def kernel(features, type_mask, emi_u0, edge_dst_u0, target_idx_u0, emi_u1, edge_dst_u1, target_idx_u1, emi_r0, edge_dst_r0, target_idx_r0, emi_r1, edge_dst_r1, target_idx_r1, attn_u0, attn_u1, attn_r0, attn_r1, r_vec, W1_u, b1_u, w2_u, W1_r, b1_r, w2_r, fcu_W, fcu_b, fcr_W, fcr_b):
    raise NotImplementedError("write your pallas kernel here")


# speedup vs baseline: 1.0742x; 1.0742x over previous
import jax
import jax.numpy as jnp
from jax.experimental import pallas as pl
from jax.experimental.pallas import tpu as pltpu

N_NODES = 10000
N_G = 10000
IN_DIM = 128
OUT_DIM = 64
NUM_HEADS = 8
ATTN_VEC_DIM = 128
E_MP = 50000
L_MP = 3
B = 4096
NUM_EDGE_TYPE = 4
_ETYPES = {'u0': (0, 1), 'u1': (1, 0), 'r0': (2, 3), 'r1': (3, 2)}

HD = NUM_HEADS * IN_DIM
_RB = 512
_NRB = B // _RB


def _r_vec_table(r_vec):
    norm = jnp.sqrt(jnp.sum(r_vec ** 2, axis=2, keepdims=True))
    rv = r_vec / jnp.maximum(norm, 1e-12)
    rv_conj = jnp.stack([rv[..., 0], -rv[..., 1]], axis=-1)
    return jnp.stack([rv, rv_conj], axis=1).reshape(NUM_EDGE_TYPE, IN_DIM // 2, 2)


def _metapath(features, r_vec_c, etypes, emi, edge_dst, target_idx, attn):
    E, L = emi.shape
    D = features.shape[1]
    edata = jnp.take(features, emi, axis=0).reshape(E, L, D // 2, 2)
    fr = [None] * L
    fr[L - 1] = jnp.stack([jnp.ones((D // 2,), jnp.float32), jnp.zeros((D // 2,), jnp.float32)], axis=-1)
    for i in range(L - 2, -1, -1):
        rv = r_vec_c[etypes[i]]
        nxt = fr[i + 1]
        re = nxt[:, 0] * rv[:, 0] - nxt[:, 1] * rv[:, 1]
        im = nxt[:, 0] * rv[:, 1] + nxt[:, 1] * rv[:, 0]
        fr[i] = jnp.stack([re, im], axis=-1)
    frv = jnp.stack(fr, axis=0)
    re = edata[..., 0] * frv[None, :, :, 0] - edata[..., 1] * frv[None, :, :, 1]
    im = edata[..., 0] * frv[None, :, :, 1] + edata[..., 1] * frv[None, :, :, 0]
    rot = jnp.stack([re, im], axis=-1).reshape(E, L, D)
    hidden = jnp.mean(rot, axis=1)
    a = hidden @ attn.T
    a = jnp.where(a >= 0, a, 0.01 * a)
    amax = jax.ops.segment_max(a, edge_dst, num_segments=N_G)
    amax = jnp.where(jnp.isfinite(amax), amax, 0.0)
    ex = jnp.exp(a - jnp.take(amax, edge_dst, axis=0))
    s = jax.ops.segment_sum(ex, edge_dst, num_segments=N_G)
    an = ex / (jnp.take(s, edge_dst, axis=0) + 1e-10)
    ft = hidden[:, None, :] * an[:, :, None]
    node_ft = jax.ops.segment_sum(ft, edge_dst, num_segments=N_G)
    ret = jnp.take(node_ft, target_idx, axis=0)
    return jax.nn.elu(ret).reshape(ret.shape[0], NUM_HEADS * D)


def _beta_body(o0, o1, o2, o3, W1u, b1u, W1r, b1r, acc_ref):
    i = pl.program_id(0)

    @pl.when(i == 0)
    def _init():
        acc_ref[...] = jnp.zeros_like(acc_ref)

    pu0 = jnp.sum(jnp.tanh(o0[...] @ W1u[...] + b1u[...][None, :]), axis=0)
    pu1 = jnp.sum(jnp.tanh(o1[...] @ W1u[...] + b1u[...][None, :]), axis=0)
    pr0 = jnp.sum(jnp.tanh(o2[...] @ W1r[...] + b1r[...][None, :]), axis=0)
    pr1 = jnp.sum(jnp.tanh(o3[...] @ W1r[...] + b1r[...][None, :]), axis=0)
    acc_ref[...] += jnp.stack([pu0, pu1, pr0, pr1], axis=0)


def _tail_body(betas, o0, o1, o2, o3, fcuW, fcub, fcrW, fcrb,
               lu_ref, lr_ref, hu_ref, hr_ref):
    bu0 = betas[0]
    bu1 = betas[1]
    br0 = betas[2]
    br1 = betas[3]
    hu = bu0 * o0[...] + bu1 * o1[...]
    hr = br0 * o2[...] + br1 * o3[...]
    hu_ref[...] = hu
    hr_ref[...] = hr
    lu_ref[...] = hu @ fcuW[...] + fcub[...][None, :]
    lr_ref[...] = hr @ fcrW[...] + fcrb[...][None, :]


def _dense_tail(outs, W1_u, b1_u, w2_u, W1_r, b1_r, w2_r, fcu_W, fcu_b, fcr_W, fcr_b):
    o0, o1, o2, o3 = outs
    row_spec = pl.BlockSpec((_RB, HD), lambda i: (i, 0))
    full = pl.BlockSpec((HD, ATTN_VEC_DIM), lambda i: (0, 0))
    vec = pl.BlockSpec((ATTN_VEC_DIM,), lambda i: (0,))
    acc = pl.pallas_call(
        _beta_body,
        grid=(_NRB,),
        in_specs=[row_spec, row_spec, row_spec, row_spec, full, vec, full, vec],
        out_specs=pl.BlockSpec((4, ATTN_VEC_DIM), lambda i: (0, 0)),
        out_shape=jax.ShapeDtypeStruct((4, ATTN_VEC_DIM), jnp.float32),
    )(o0, o1, o2, o3, W1_u, b1_u, W1_r, b1_r)
    mean = acc / B
    beta_u = jax.nn.softmax((mean[0:2] @ w2_u)[:, 0])
    beta_r = jax.nn.softmax((mean[2:4] @ w2_r)[:, 0])
    betas = jnp.concatenate([beta_u, beta_r], axis=0)

    fcW_spec = pl.BlockSpec((HD, OUT_DIM), lambda i: (0, 0))
    fcb_spec = pl.BlockSpec((OUT_DIM,), lambda i: (0,))
    lu, lr, hu, hr = pl.pallas_call(
        _tail_body,
        grid=(_NRB,),
        in_specs=[pl.BlockSpec(memory_space=pltpu.SMEM),
                  row_spec, row_spec, row_spec, row_spec,
                  fcW_spec, fcb_spec, fcW_spec, fcb_spec],
        out_specs=[pl.BlockSpec((_RB, OUT_DIM), lambda i: (i, 0)),
                   pl.BlockSpec((_RB, OUT_DIM), lambda i: (i, 0)),
                   row_spec, row_spec],
        out_shape=[jax.ShapeDtypeStruct((B, OUT_DIM), jnp.float32),
                   jax.ShapeDtypeStruct((B, OUT_DIM), jnp.float32),
                   jax.ShapeDtypeStruct((B, HD), jnp.float32),
                   jax.ShapeDtypeStruct((B, HD), jnp.float32)],
    )(betas, o0, o1, o2, o3, fcu_W, fcu_b, fcr_W, fcr_b)
    return lu, lr, hu, hr


def kernel(features, type_mask, emi_u0, edge_dst_u0, target_idx_u0, emi_u1, edge_dst_u1, target_idx_u1, emi_r0, edge_dst_r0, target_idx_r0, emi_r1, edge_dst_r1, target_idx_r1, attn_u0, attn_u1, attn_r0, attn_r1, r_vec, W1_u, b1_u, w2_u, W1_r, b1_r, w2_r, fcu_W, fcu_b, fcr_W, fcr_b):
    del type_mask
    r_vec_c = _r_vec_table(r_vec)
    out_u0 = _metapath(features, r_vec_c, _ETYPES['u0'], emi_u0, edge_dst_u0, target_idx_u0, attn_u0)
    out_u1 = _metapath(features, r_vec_c, _ETYPES['u1'], emi_u1, edge_dst_u1, target_idx_u1, attn_u1)
    out_r0 = _metapath(features, r_vec_c, _ETYPES['r0'], emi_r0, edge_dst_r0, target_idx_r0, attn_r0)
    out_r1 = _metapath(features, r_vec_c, _ETYPES['r1'], emi_r1, edge_dst_r1, target_idx_r1, attn_r1)
    return _dense_tail([out_u0, out_u1, out_r0, out_r1],
                       W1_u, b1_u, w2_u, W1_r, b1_r, w2_r,
                       fcu_W, fcu_b, fcr_W, fcr_b)
